# Initial kernel scaffold; baseline (speedup 1.0000x reference)
import functools

import jax
import jax.numpy as jnp
from jax import lax
from jax.experimental import pallas as pl
from jax.experimental.pallas import tpu as pltpu
from jax.experimental.pallas import tpu_sc as plsc

N = 10000
E = 160000
D = 128

NC = 2
NS = 16
EPC = E // NC
EPT = EPC // NS
CH = 128
NFULL = EPT // CH
TAIL = EPT - NFULL * CH
RPT = N // NS

_f32 = jnp.float32


def _lrelu(x):
    return jnp.where(x >= 0, x, 0.01 * x)


def _mm_rows(x, w, act, block_rows):
    R, K = x.shape
    C = w.shape[1]

    def body(x_ref, w_ref, o_ref):
        y = jnp.dot(x_ref[...], w_ref[...], preferred_element_type=_f32)
        o_ref[...] = _lrelu(y) if act else y

    return pl.pallas_call(
        body,
        grid=(R // block_rows,),
        in_specs=[
            pl.BlockSpec((block_rows, K), lambda i: (i, 0)),
            pl.BlockSpec((K, C), lambda i: (0, 0)),
        ],
        out_specs=pl.BlockSpec((block_rows, C), lambda i: (i, 0)),
        out_shape=jax.ShapeDtypeStruct((R, C), _f32),
    )(x, w)


def _mm_sum2(m0, m1, w, block_rows):
    R, K = m0.shape
    C = w.shape[1]

    def body(a_ref, b_ref, w_ref, o_ref):
        s = a_ref[...] + b_ref[...]
        o_ref[...] = jnp.dot(s, w_ref[...], preferred_element_type=_f32)

    spec = pl.BlockSpec((block_rows, K), lambda i: (i, 0))
    return pl.pallas_call(
        body,
        grid=(R // block_rows,),
        in_specs=[spec, spec, pl.BlockSpec((K, C), lambda i: (0, 0))],
        out_specs=pl.BlockSpec((block_rows, C), lambda i: (i, 0)),
        out_shape=jax.ShapeDtypeStruct((R, C), _f32),
    )(m0, m1, w)


def _layer_update(n0, n1, ne, efe, w2, block_rows):
    R, K = ne.shape

    def body(n0_ref, n1_ref, ne_ref, efe_ref, w_ref, o_ref):
        s = n0_ref[...] + n1_ref[...]
        y = ne_ref[...] + efe_ref[...] + jnp.dot(
            s, w_ref[...], preferred_element_type=_f32)
        o_ref[...] = _lrelu(y)

    spec = pl.BlockSpec((block_rows, K), lambda i: (i, 0))
    return pl.pallas_call(
        body,
        grid=(R // block_rows,),
        in_specs=[spec, spec, spec, spec,
                  pl.BlockSpec((K, K), lambda i: (0, 0))],
        out_specs=pl.BlockSpec((block_rows, K), lambda i: (i, 0)),
        out_shape=jax.ShapeDtypeStruct((R, K), _f32),
    )(n0, n1, ne, efe, w2)


def _post(emb, w7, w5, block_rows):
    R = emb.shape[0]

    def body(emb_ref, w7_ref, w5_ref, a_ref, b_ref, g_ref):
        i = pl.program_id(0)
        npj = jnp.dot(emb_ref[...], w7_ref[...], preferred_element_type=_f32)
        lr = _lrelu(npj)
        a_ref[...] = jnp.dot(lr, w5_ref[D:2 * D, :],
                             preferred_element_type=_f32)
        b_ref[...] = jnp.dot(lr, w5_ref[2 * D:3 * D, :],
                             preferred_element_type=_f32)

        @pl.when(i == 0)
        def _():
            g_ref[...] = jnp.zeros_like(g_ref)

        g_ref[...] += jnp.sum(emb_ref[...], axis=0, keepdims=True)

    return pl.pallas_call(
        body,
        grid=(R // block_rows,),
        in_specs=[
            pl.BlockSpec((block_rows, D), lambda i: (i, 0)),
            pl.BlockSpec((D, D), lambda i: (0, 0)),
            pl.BlockSpec((3 * D, 1), lambda i: (0, 0)),
        ],
        out_specs=[
            pl.BlockSpec((block_rows, 1), lambda i: (i, 0)),
            pl.BlockSpec((block_rows, 1), lambda i: (i, 0)),
            pl.BlockSpec((1, D), lambda i: (0, 0)),
        ],
        out_shape=[
            jax.ShapeDtypeStruct((R, 1), _f32),
            jax.ShapeDtypeStruct((R, 1), _f32),
            jax.ShapeDtypeStruct((1, D), _f32),
        ],
    )(emb, w7, w5)


def _finalize(g, w6, w5, wnoop):
    def body(g_ref, w6_ref, w5_ref, wn_ref, c_ref, noop_ref):
        gv = g_ref[...]
        lr = _lrelu(jnp.dot(gv, w6_ref[...], preferred_element_type=_f32))
        c = jnp.dot(lr, w5_ref[0:D, :], preferred_element_type=_f32)
        c_ref[...] = jnp.broadcast_to(c, c_ref.shape)
        noop_ref[...] = jnp.dot(gv, wn_ref[...], preferred_element_type=_f32)

    return pl.pallas_call(
        body,
        out_shape=[
            jax.ShapeDtypeStruct((1, D), _f32),
            jax.ShapeDtypeStruct((1, 1), _f32),
        ],
    )(g, w6, w5, wnoop)


_MESH = plsc.VectorSubcoreMesh(core_axis_name="c", subcore_axis_name="s")


def _sc_scatter_rows(rows_hbm, u_hbm, v_hbm, zeros_hbm):
    @functools.partial(
        pl.kernel,
        out_type=[jax.ShapeDtypeStruct((N, D), _f32),
                  jax.ShapeDtypeStruct((N, D), _f32)],
        mesh=_MESH,
        scratch_types=[
            pltpu.VMEM((CH,), jnp.int32),
            pltpu.VMEM((CH,), jnp.int32),
            pltpu.VMEM((CH, D), _f32),
            pltpu.VMEM((TAIL,), jnp.int32),
            pltpu.VMEM((TAIL,), jnp.int32),
            pltpu.VMEM((TAIL, D), _f32),
            pltpu.VMEM_SHARED((N, D), _f32),
        ],
    )
    def k(rows_h, u_h, v_h, z_h, out0, out1, iu, iv, rows, iu8, iv8, rows8,
          acc):
        cid = lax.axis_index("c")
        sid = lax.axis_index("s")
        r0 = sid * RPT
        pltpu.sync_copy(z_h.at[pl.ds(r0, RPT)], acc.at[pl.ds(r0, RPT)])
        plsc.subcore_barrier()

        base = cid * EPC + sid * EPT

        def chunk(kk, _):
            off = base + kk * CH
            pltpu.sync_copy(u_h.at[pl.ds(off, CH)], iu)
            pltpu.sync_copy(v_h.at[pl.ds(off, CH)], iv)
            pltpu.sync_copy(rows_h.at[pl.ds(off, CH)], rows)
            pltpu.sync_copy(rows, acc.at[iu], add=True)
            pltpu.sync_copy(rows, acc.at[iv], add=True)
            return 0

        lax.fori_loop(0, NFULL, chunk, 0)
        toff = base + NFULL * CH
        pltpu.sync_copy(u_h.at[pl.ds(toff, TAIL)], iu8)
        pltpu.sync_copy(v_h.at[pl.ds(toff, TAIL)], iv8)
        pltpu.sync_copy(rows_h.at[pl.ds(toff, TAIL)], rows8)
        pltpu.sync_copy(rows8, acc.at[iu8], add=True)
        pltpu.sync_copy(rows8, acc.at[iv8], add=True)

        plsc.subcore_barrier()

        @pl.when(cid == 0)
        def _():
            pltpu.sync_copy(acc.at[pl.ds(r0, RPT)], out0.at[pl.ds(r0, RPT)])

        @pl.when(cid == 1)
        def _():
            pltpu.sync_copy(acc.at[pl.ds(r0, RPT)], out1.at[pl.ds(r0, RPT)])

    return k(rows_hbm, u_hbm, v_hbm, zeros_hbm)


def _sc_neighbor_sum(emb_hbm, u_hbm, v_hbm, zeros_hbm):
    @functools.partial(
        pl.kernel,
        out_type=[jax.ShapeDtypeStruct((N, D), _f32),
                  jax.ShapeDtypeStruct((N, D), _f32)],
        mesh=_MESH,
        scratch_types=[
            pltpu.VMEM((CH,), jnp.int32),
            pltpu.VMEM((CH,), jnp.int32),
            pltpu.VMEM((CH, D), _f32),
            pltpu.VMEM((CH, D), _f32),
            pltpu.VMEM((TAIL,), jnp.int32),
            pltpu.VMEM((TAIL,), jnp.int32),
            pltpu.VMEM((TAIL, D), _f32),
            pltpu.VMEM((TAIL, D), _f32),
            pltpu.VMEM_SHARED((N, D), _f32),
            pltpu.SemaphoreType.DMA,
        ],
    )
    def k(emb_h, u_h, v_h, z_h, out0, out1, iu, iv, ru, rv, iu8, iv8, ru8,
          rv8, acc, sem):
        cid = lax.axis_index("c")
        sid = lax.axis_index("s")
        r0 = sid * RPT
        pltpu.sync_copy(z_h.at[pl.ds(r0, RPT)], acc.at[pl.ds(r0, RPT)])
        plsc.subcore_barrier()

        base = cid * EPC + sid * EPT

        def chunk(kk, _):
            off = base + kk * CH
            pltpu.sync_copy(u_h.at[pl.ds(off, CH)], iu)
            pltpu.sync_copy(v_h.at[pl.ds(off, CH)], iv)
            pltpu.async_copy(emb_h.at[iv], rv, sem).wait()
            pltpu.sync_copy(rv, acc.at[iu], add=True)
            pltpu.async_copy(emb_h.at[iu], ru, sem).wait()
            pltpu.sync_copy(ru, acc.at[iv], add=True)
            return 0

        lax.fori_loop(0, NFULL, chunk, 0)
        toff = base + NFULL * CH
        pltpu.sync_copy(u_h.at[pl.ds(toff, TAIL)], iu8)
        pltpu.sync_copy(v_h.at[pl.ds(toff, TAIL)], iv8)
        pltpu.async_copy(emb_h.at[iv8], rv8, sem).wait()
        pltpu.sync_copy(rv8, acc.at[iu8], add=True)
        pltpu.async_copy(emb_h.at[iu8], ru8, sem).wait()
        pltpu.sync_copy(ru8, acc.at[iv8], add=True)

        plsc.subcore_barrier()

        @pl.when(cid == 0)
        def _():
            pltpu.sync_copy(acc.at[pl.ds(r0, RPT)], out0.at[pl.ds(r0, RPT)])

        @pl.when(cid == 1)
        def _():
            pltpu.sync_copy(acc.at[pl.ds(r0, RPT)], out1.at[pl.ds(r0, RPT)])

    return k(emb_hbm, u_hbm, v_hbm, zeros_hbm)


_QV = EPT // 16
_QTAIL = EPT - _QV * 16
_UPAD = _QV * 16 + 16


def _sc_edge_q(a_hbm, b_hbm, c_hbm, u_hbm, v_hbm):
    @functools.partial(
        pl.kernel,
        out_type=jax.ShapeDtypeStruct((E,), _f32),
        mesh=_MESH,
        scratch_types=[
            pltpu.VMEM((N,), _f32),
            pltpu.VMEM((N,), _f32),
            pltpu.VMEM((16,), _f32),
            pltpu.VMEM((_UPAD,), jnp.int32),
            pltpu.VMEM((_UPAD,), jnp.int32),
            pltpu.VMEM((_UPAD,), _f32),
        ],
    )
    def k(a_h, b_h, c_h, u_h, v_h, out, abuf, bbuf, cbuf, ubuf, vbuf, qbuf):
        cid = lax.axis_index("c")
        sid = lax.axis_index("s")
        tid = cid * NS + sid
        base = tid * EPT
        pltpu.sync_copy(a_h, abuf)
        pltpu.sync_copy(b_h, bbuf)
        pltpu.sync_copy(c_h, cbuf)
        pltpu.sync_copy(u_h.at[pl.ds(base, EPT)], ubuf.at[pl.ds(0, EPT)])
        pltpu.sync_copy(v_h.at[pl.ds(base, EPT)], vbuf.at[pl.ds(0, EPT)])
        cv = cbuf[...]

        def body(i, _):
            idxu = ubuf[pl.ds(i * 16, 16)]
            idxv = vbuf[pl.ds(i * 16, 16)]
            av = plsc.load_gather(abuf, [idxu])
            bv = plsc.load_gather(bbuf, [idxv])
            qbuf[pl.ds(i * 16, 16)] = av + bv + cv
            return 0

        lax.fori_loop(0, _QV, body, 0)
        msk = lax.iota(jnp.int32, 16) < _QTAIL
        idxu = ubuf[pl.ds(_QV * 16, 16)]
        idxv = vbuf[pl.ds(_QV * 16, 16)]
        av = plsc.load_gather(abuf, [idxu], mask=msk)
        bv = plsc.load_gather(bbuf, [idxv], mask=msk)
        qbuf[pl.ds(_QV * 16, 16)] = av + bv + cv

        pltpu.sync_copy(qbuf.at[pl.ds(0, EPT)], out.at[pl.ds(base, EPT)])

    return k(a_hbm, b_hbm, c_hbm, u_hbm, v_hbm)


def kernel(state, edge_features, edges_ij, W1, W2, W3, W4, W5, W6, W7, Wnoop):
    u = edges_ij[:, 0]
    v = edges_ij[:, 1]
    zeros_nd = jnp.zeros((N, D), _f32)

    ne = _mm_rows(state[0], W1, act=False, block_rows=2000)
    x4 = _mm_rows(edge_features[0], W4, act=True, block_rows=4000)

    m0, m1 = _sc_scatter_rows(x4, u, v, zeros_nd)
    efe = _mm_sum2(m0, m1, W3, block_rows=2000)

    emb = ne
    for _ in range(3):
        n0, n1 = _sc_neighbor_sum(emb, u, v, zeros_nd)
        emb = _layer_update(n0, n1, ne, efe, W2, block_rows=2000)

    a, b, g = _post(emb, W7, W5, block_rows=2000)
    c_full, noop = _finalize(g, W6, W5, Wnoop)
    c16 = c_full[0, :16]

    eq = _sc_edge_q(a[:, 0], b[:, 0], c16, u, v)
    return jnp.concatenate([eq[None, :], noop], axis=1)


# speedup vs baseline: 4.6060x; 4.6060x over previous
import functools

import jax
import jax.numpy as jnp
from jax import lax
from jax.experimental import pallas as pl
from jax.experimental.pallas import tpu as pltpu
from jax.experimental.pallas import tpu_sc as plsc

N = 10000
E = 160000
D = 128

NC = 2
NS = 16
EPC = E // NC
EPT = EPC // NS
CH = 128
NFULL = EPT // CH
TAIL = EPT - NFULL * CH
RPT = 624
RREM = N - NS * RPT

_f32 = jnp.float32


def _lrelu(x):
    return jnp.where(x >= 0, x, 0.01 * x)


def _mm_rows(x, w, act, block_rows):
    R, K = x.shape
    C = w.shape[1]

    def body(x_ref, w_ref, o_ref):
        y = jnp.dot(x_ref[...], w_ref[...], preferred_element_type=_f32)
        o_ref[...] = _lrelu(y) if act else y

    return pl.pallas_call(
        body,
        grid=(R // block_rows,),
        in_specs=[
            pl.BlockSpec((block_rows, K), lambda i: (i, 0)),
            pl.BlockSpec((K, C), lambda i: (0, 0)),
        ],
        out_specs=pl.BlockSpec((block_rows, C), lambda i: (i, 0)),
        out_shape=jax.ShapeDtypeStruct((R, C), _f32),
    )(x, w)


def _mm_sum2(m0, m1, w, block_rows):
    R, K = m0.shape
    C = w.shape[1]

    def body(a_ref, b_ref, w_ref, o_ref):
        s = a_ref[...] + b_ref[...]
        o_ref[...] = jnp.dot(s, w_ref[...], preferred_element_type=_f32)

    spec = pl.BlockSpec((block_rows, K), lambda i: (i, 0))
    return pl.pallas_call(
        body,
        grid=(R // block_rows,),
        in_specs=[spec, spec, pl.BlockSpec((K, C), lambda i: (0, 0))],
        out_specs=pl.BlockSpec((block_rows, C), lambda i: (i, 0)),
        out_shape=jax.ShapeDtypeStruct((R, C), _f32),
    )(m0, m1, w)


def _layer_update(n0, n1, ne, efe, w2, block_rows):
    R, K = ne.shape

    def body(n0_ref, n1_ref, ne_ref, efe_ref, w_ref, o_ref):
        s = n0_ref[...] + n1_ref[...]
        y = ne_ref[...] + efe_ref[...] + jnp.dot(
            s, w_ref[...], preferred_element_type=_f32)
        o_ref[...] = _lrelu(y)

    spec = pl.BlockSpec((block_rows, K), lambda i: (i, 0))
    return pl.pallas_call(
        body,
        grid=(R // block_rows,),
        in_specs=[spec, spec, spec, spec,
                  pl.BlockSpec((K, K), lambda i: (0, 0))],
        out_specs=pl.BlockSpec((block_rows, K), lambda i: (i, 0)),
        out_shape=jax.ShapeDtypeStruct((R, K), _f32),
    )(n0, n1, ne, efe, w2)


def _post(emb, w7, w5, block_rows):
    R = emb.shape[0]

    def body(emb_ref, w7_ref, w5_ref, a_ref, b_ref, g_ref):
        i = pl.program_id(0)
        npj = jnp.dot(emb_ref[...], w7_ref[...], preferred_element_type=_f32)
        lr = _lrelu(npj)
        a_ref[...] = jnp.dot(lr, w5_ref[D:2 * D, :],
                             preferred_element_type=_f32)
        b_ref[...] = jnp.dot(lr, w5_ref[2 * D:3 * D, :],
                             preferred_element_type=_f32)

        @pl.when(i == 0)
        def _():
            g_ref[...] = jnp.zeros_like(g_ref)

        g_ref[...] += jnp.sum(emb_ref[...], axis=0, keepdims=True)

    return pl.pallas_call(
        body,
        grid=(R // block_rows,),
        in_specs=[
            pl.BlockSpec((block_rows, D), lambda i: (i, 0)),
            pl.BlockSpec((D, D), lambda i: (0, 0)),
            pl.BlockSpec((3 * D, 1), lambda i: (0, 0)),
        ],
        out_specs=[
            pl.BlockSpec((block_rows, 1), lambda i: (i, 0)),
            pl.BlockSpec((block_rows, 1), lambda i: (i, 0)),
            pl.BlockSpec((1, D), lambda i: (0, 0)),
        ],
        out_shape=[
            jax.ShapeDtypeStruct((R, 1), _f32),
            jax.ShapeDtypeStruct((R, 1), _f32),
            jax.ShapeDtypeStruct((1, D), _f32),
        ],
    )(emb, w7, w5)


def _finalize(g, w6, w5, wnoop):
    def body(g_ref, w6_ref, w5_ref, wn_ref, c_ref, noop_ref):
        gv = g_ref[...]
        lr = _lrelu(jnp.dot(gv, w6_ref[...], preferred_element_type=_f32))
        c = jnp.dot(lr, w5_ref[0:D, :], preferred_element_type=_f32)
        c_ref[...] = jnp.broadcast_to(c, c_ref.shape)
        noop_ref[...] = jnp.dot(gv, wn_ref[...], preferred_element_type=_f32)

    return pl.pallas_call(
        body,
        out_shape=[
            jax.ShapeDtypeStruct((1, D), _f32),
            jax.ShapeDtypeStruct((1, 1), _f32),
        ],
    )(g, w6, w5, wnoop)


_MESH = plsc.VectorSubcoreMesh(core_axis_name="c", subcore_axis_name="s")


def _tile_rows_copy(src, dst, sid):
    r0 = sid * RPT
    pltpu.sync_copy(src.at[pl.ds(r0, RPT)], dst.at[pl.ds(r0, RPT)])

    @pl.when(sid == 0)
    def _():
        pltpu.sync_copy(src.at[pl.ds(NS * RPT, RREM)],
                        dst.at[pl.ds(NS * RPT, RREM)])


def _sc_scatter_rows(rows_hbm, u_hbm, v_hbm, zeros_hbm):
    @functools.partial(
        pl.kernel,
        out_type=[jax.ShapeDtypeStruct((N, D), _f32),
                  jax.ShapeDtypeStruct((N, D), _f32)],
        mesh=_MESH,
        scratch_types=[
            pltpu.VMEM((CH,), jnp.int32),
            pltpu.VMEM((CH,), jnp.int32),
            pltpu.VMEM((CH, D), _f32),
            pltpu.VMEM((TAIL,), jnp.int32),
            pltpu.VMEM((TAIL,), jnp.int32),
            pltpu.VMEM((TAIL, D), _f32),
            pltpu.VMEM_SHARED((N, D), _f32),
        ],
    )
    def k(rows_h, u_h, v_h, z_h, out0, out1, iu, iv, rows, iu8, iv8, rows8,
          acc):
        cid = lax.axis_index("c")
        sid = lax.axis_index("s")
        _tile_rows_copy(z_h, acc, sid)
        plsc.subcore_barrier()

        base = cid * EPC + sid * EPT

        def chunk(kk, _):
            off = base + kk * CH
            pltpu.sync_copy(u_h.at[pl.ds(off, CH)], iu)
            pltpu.sync_copy(v_h.at[pl.ds(off, CH)], iv)
            pltpu.sync_copy(rows_h.at[pl.ds(off, CH)], rows)
            pltpu.sync_copy(rows, acc.at[iu], add=True)
            pltpu.sync_copy(rows, acc.at[iv], add=True)
            return 0

        lax.fori_loop(0, NFULL, chunk, 0)
        toff = base + NFULL * CH
        pltpu.sync_copy(u_h.at[pl.ds(toff, TAIL)], iu8)
        pltpu.sync_copy(v_h.at[pl.ds(toff, TAIL)], iv8)
        pltpu.sync_copy(rows_h.at[pl.ds(toff, TAIL)], rows8)
        pltpu.sync_copy(rows8, acc.at[iu8], add=True)
        pltpu.sync_copy(rows8, acc.at[iv8], add=True)

        plsc.subcore_barrier()

        @pl.when(cid == 0)
        def _():
            _tile_rows_copy(acc, out0, sid)

        @pl.when(cid == 1)
        def _():
            _tile_rows_copy(acc, out1, sid)

    return k(rows_hbm, u_hbm, v_hbm, zeros_hbm)


def _sc_neighbor_sum(emb_hbm, u_hbm, v_hbm, zeros_hbm):
    @functools.partial(
        pl.kernel,
        out_type=[jax.ShapeDtypeStruct((N, D), _f32),
                  jax.ShapeDtypeStruct((N, D), _f32)],
        mesh=_MESH,
        scratch_types=[
            pltpu.VMEM((CH,), jnp.int32),
            pltpu.VMEM((CH,), jnp.int32),
            pltpu.VMEM((CH, D), _f32),
            pltpu.VMEM((CH, D), _f32),
            pltpu.VMEM((TAIL,), jnp.int32),
            pltpu.VMEM((TAIL,), jnp.int32),
            pltpu.VMEM((TAIL, D), _f32),
            pltpu.VMEM((TAIL, D), _f32),
            pltpu.VMEM_SHARED((N, D), _f32),
            pltpu.SemaphoreType.DMA,
        ],
    )
    def k(emb_h, u_h, v_h, z_h, out0, out1, iu, iv, ru, rv, iu8, iv8, ru8,
          rv8, acc, sem):
        cid = lax.axis_index("c")
        sid = lax.axis_index("s")
        _tile_rows_copy(z_h, acc, sid)
        plsc.subcore_barrier()

        base = cid * EPC + sid * EPT

        def chunk(kk, _):
            off = base + kk * CH
            pltpu.sync_copy(u_h.at[pl.ds(off, CH)], iu)
            pltpu.sync_copy(v_h.at[pl.ds(off, CH)], iv)
            pltpu.async_copy(emb_h.at[iv], rv, sem).wait()
            pltpu.sync_copy(rv, acc.at[iu], add=True)
            pltpu.async_copy(emb_h.at[iu], ru, sem).wait()
            pltpu.sync_copy(ru, acc.at[iv], add=True)
            return 0

        lax.fori_loop(0, NFULL, chunk, 0)
        toff = base + NFULL * CH
        pltpu.sync_copy(u_h.at[pl.ds(toff, TAIL)], iu8)
        pltpu.sync_copy(v_h.at[pl.ds(toff, TAIL)], iv8)
        pltpu.async_copy(emb_h.at[iv8], rv8, sem).wait()
        pltpu.sync_copy(rv8, acc.at[iu8], add=True)
        pltpu.async_copy(emb_h.at[iu8], ru8, sem).wait()
        pltpu.sync_copy(ru8, acc.at[iv8], add=True)

        plsc.subcore_barrier()

        @pl.when(cid == 0)
        def _():
            _tile_rows_copy(acc, out0, sid)

        @pl.when(cid == 1)
        def _():
            _tile_rows_copy(acc, out1, sid)

    return k(emb_hbm, u_hbm, v_hbm, zeros_hbm)


_EPAD = EPT + 16 - EPT % 16
_QG = _EPAD // 16


def _sc_edge_q(a_hbm, b_hbm, c_hbm, u_hbm, v_hbm):
    @functools.partial(
        pl.kernel,
        out_type=jax.ShapeDtypeStruct((E,), _f32),
        mesh=_MESH,
        scratch_types=[
            pltpu.VMEM((16,), _f32),
            pltpu.VMEM((EPT,), jnp.int32),
            pltpu.VMEM((EPT,), jnp.int32),
            pltpu.VMEM((_EPAD,), _f32),
            pltpu.VMEM((_EPAD,), _f32),
            pltpu.VMEM((_EPAD,), _f32),
            pltpu.SemaphoreType.DMA,
        ],
    )
    def k(a_h, b_h, c_h, u_h, v_h, out, cbuf, ubuf, vbuf, av, bv, qbuf, sem):
        cid = lax.axis_index("c")
        sid = lax.axis_index("s")
        tid = cid * NS + sid
        base = tid * EPT
        pltpu.sync_copy(c_h, cbuf)
        pltpu.sync_copy(u_h.at[pl.ds(base, EPT)], ubuf)
        pltpu.sync_copy(v_h.at[pl.ds(base, EPT)], vbuf)
        cp_a = pltpu.async_copy(a_h.at[ubuf], av.at[pl.ds(0, EPT)], sem)
        cp_b = pltpu.async_copy(b_h.at[vbuf], bv.at[pl.ds(0, EPT)], sem)
        cp_a.wait()
        cp_b.wait()
        cv = cbuf[...]

        def body(i, _):
            qbuf[pl.ds(i * 16, 16)] = (
                av[pl.ds(i * 16, 16)] + bv[pl.ds(i * 16, 16)] + cv)
            return 0

        lax.fori_loop(0, _QG, body, 0)
        pltpu.sync_copy(qbuf.at[pl.ds(0, EPT)], out.at[pl.ds(base, EPT)])

    return k(a_hbm, b_hbm, c_hbm, u_hbm, v_hbm)


def kernel(state, edge_features, edges_ij, W1, W2, W3, W4, W5, W6, W7, Wnoop):
    u = edges_ij[:, 0]
    v = edges_ij[:, 1]
    zeros_nd = jnp.zeros((N, D), _f32)

    ne = _mm_rows(state[0], W1, act=False, block_rows=2000)
    x4 = _mm_rows(edge_features[0], W4, act=True, block_rows=4000)

    m0, m1 = _sc_scatter_rows(x4, u, v, zeros_nd)
    efe = _mm_sum2(m0, m1, W3, block_rows=2000)

    emb = ne
    for _ in range(3):
        n0, n1 = _sc_neighbor_sum(emb, u, v, zeros_nd)
        emb = _layer_update(n0, n1, ne, efe, W2, block_rows=2000)

    a, b, g = _post(emb, W7, W5, block_rows=2000)
    c_full, noop = _finalize(g, W6, W5, Wnoop)
    c16 = c_full[0, :16]

    eq = _sc_edge_q(a[:, 0], b[:, 0], c16, u, v)
    return jnp.concatenate([eq[None, :], noop], axis=1)


# speedup vs baseline: 7.4937x; 1.6269x over previous
import functools

import jax
import jax.numpy as jnp
from jax import lax
from jax.experimental import pallas as pl
from jax.experimental.pallas import tpu as pltpu
from jax.experimental.pallas import tpu_sc as plsc

N = 10000
E = 160000
D = 128

NC = 2
NS = 16
EPC = E // NC
EPT = EPC // NS
CH = 128
NFULL = EPT // CH
TAIL = EPT - NFULL * CH
CHN = 64
NFULLN = EPT // CHN
NBUF = 2
RPT = 624
RREM = N - NS * RPT

_f32 = jnp.float32


def _lrelu(x):
    return jnp.where(x >= 0, x, 0.01 * x)


def _mm_rows(x, w, act, block_rows):
    R, K = x.shape
    C = w.shape[1]

    def body(x_ref, w_ref, o_ref):
        y = jnp.dot(x_ref[...], w_ref[...], preferred_element_type=_f32)
        o_ref[...] = _lrelu(y) if act else y

    return pl.pallas_call(
        body,
        grid=(R // block_rows,),
        in_specs=[
            pl.BlockSpec((block_rows, K), lambda i: (i, 0)),
            pl.BlockSpec((K, C), lambda i: (0, 0)),
        ],
        out_specs=pl.BlockSpec((block_rows, C), lambda i: (i, 0)),
        out_shape=jax.ShapeDtypeStruct((R, C), _f32),
    )(x, w)


def _mm_sum2(m0, m1, w, block_rows):
    R, K = m0.shape
    C = w.shape[1]

    def body(a_ref, b_ref, w_ref, o_ref):
        s = a_ref[...] + b_ref[...]
        o_ref[...] = jnp.dot(s, w_ref[...], preferred_element_type=_f32)

    spec = pl.BlockSpec((block_rows, K), lambda i: (i, 0))
    return pl.pallas_call(
        body,
        grid=(R // block_rows,),
        in_specs=[spec, spec, pl.BlockSpec((K, C), lambda i: (0, 0))],
        out_specs=pl.BlockSpec((block_rows, C), lambda i: (i, 0)),
        out_shape=jax.ShapeDtypeStruct((R, C), _f32),
    )(m0, m1, w)


def _layer_update(n0, n1, ne, efe, w2, block_rows):
    R, K = ne.shape

    def body(n0_ref, n1_ref, ne_ref, efe_ref, w_ref, o_ref):
        s = n0_ref[...] + n1_ref[...]
        y = ne_ref[...] + efe_ref[...] + jnp.dot(
            s, w_ref[...], preferred_element_type=_f32)
        o_ref[...] = _lrelu(y)

    spec = pl.BlockSpec((block_rows, K), lambda i: (i, 0))
    return pl.pallas_call(
        body,
        grid=(R // block_rows,),
        in_specs=[spec, spec, spec, spec,
                  pl.BlockSpec((K, K), lambda i: (0, 0))],
        out_specs=pl.BlockSpec((block_rows, K), lambda i: (i, 0)),
        out_shape=jax.ShapeDtypeStruct((R, K), _f32),
    )(n0, n1, ne, efe, w2)


def _post(emb, w7, w5, block_rows):
    R = emb.shape[0]

    def body(emb_ref, w7_ref, w5_ref, a_ref, b_ref, g_ref):
        i = pl.program_id(0)
        npj = jnp.dot(emb_ref[...], w7_ref[...], preferred_element_type=_f32)
        lr = _lrelu(npj)
        a_ref[...] = jnp.dot(lr, w5_ref[D:2 * D, :],
                             preferred_element_type=_f32)
        b_ref[...] = jnp.dot(lr, w5_ref[2 * D:3 * D, :],
                             preferred_element_type=_f32)

        @pl.when(i == 0)
        def _():
            g_ref[...] = jnp.zeros_like(g_ref)

        g_ref[...] += jnp.sum(emb_ref[...], axis=0, keepdims=True)

    return pl.pallas_call(
        body,
        grid=(R // block_rows,),
        in_specs=[
            pl.BlockSpec((block_rows, D), lambda i: (i, 0)),
            pl.BlockSpec((D, D), lambda i: (0, 0)),
            pl.BlockSpec((3 * D, 1), lambda i: (0, 0)),
        ],
        out_specs=[
            pl.BlockSpec((block_rows, 1), lambda i: (i, 0)),
            pl.BlockSpec((block_rows, 1), lambda i: (i, 0)),
            pl.BlockSpec((1, D), lambda i: (0, 0)),
        ],
        out_shape=[
            jax.ShapeDtypeStruct((R, 1), _f32),
            jax.ShapeDtypeStruct((R, 1), _f32),
            jax.ShapeDtypeStruct((1, D), _f32),
        ],
    )(emb, w7, w5)


def _finalize(g, w6, w5, wnoop):
    def body(g_ref, w6_ref, w5_ref, wn_ref, c_ref, noop_ref):
        gv = g_ref[...]
        lr = _lrelu(jnp.dot(gv, w6_ref[...], preferred_element_type=_f32))
        c = jnp.dot(lr, w5_ref[0:D, :], preferred_element_type=_f32)
        c_ref[...] = jnp.broadcast_to(c, c_ref.shape)
        noop_ref[...] = jnp.dot(gv, wn_ref[...], preferred_element_type=_f32)

    return pl.pallas_call(
        body,
        out_shape=[
            jax.ShapeDtypeStruct((1, D), _f32),
            jax.ShapeDtypeStruct((1, 1), _f32),
        ],
    )(g, w6, w5, wnoop)


_MESH = plsc.VectorSubcoreMesh(core_axis_name="c", subcore_axis_name="s")


def _tile_rows_copy(src, dst, sid):
    r0 = sid * RPT
    pltpu.sync_copy(src.at[pl.ds(r0, RPT)], dst.at[pl.ds(r0, RPT)])

    @pl.when(sid == 0)
    def _():
        pltpu.sync_copy(src.at[pl.ds(NS * RPT, RREM)],
                        dst.at[pl.ds(NS * RPT, RREM)])


def _sc_scatter_rows(rows_hbm, u_hbm, v_hbm, zeros_hbm):
    @functools.partial(
        pl.kernel,
        out_type=[jax.ShapeDtypeStruct((N, D), _f32),
                  jax.ShapeDtypeStruct((N, D), _f32)],
        mesh=_MESH,
        scratch_types=[
            pltpu.VMEM((EPT,), jnp.int32),
            pltpu.VMEM((EPT,), jnp.int32),
            pltpu.VMEM((CH, D), _f32),
            pltpu.VMEM((CH, D), _f32),
            pltpu.VMEM((TAIL, D), _f32),
            pltpu.VMEM_SHARED((N, D), _f32),
            pltpu.SemaphoreType.DMA,
            pltpu.SemaphoreType.DMA,
            pltpu.SemaphoreType.DMA,
            pltpu.SemaphoreType.DMA,
        ],
    )
    def k(rows_h, u_h, v_h, z_h, out0, out1, ubuf, vbuf, r0, r1,
          rows8, acc, g0, g1, s0, s1):
        cid = lax.axis_index("c")
        sid = lax.axis_index("s")
        base = cid * EPC + sid * EPT
        rbuf = [r0, r1]
        gsem = [g0, g1]
        ssem = [s0, s1]

        pltpu.sync_copy(u_h.at[pl.ds(base, EPT)], ubuf)
        pltpu.sync_copy(v_h.at[pl.ds(base, EPT)], vbuf)
        hg = [pltpu.async_copy(rows_h.at[pl.ds(base + b * CH, CH)],
                               rbuf[b], gsem[b]) for b in range(NBUF)]
        _tile_rows_copy(z_h, acc, sid)
        plsc.subcore_barrier()

        hs = [None] * NBUF
        for kk in range(NFULL):
            b = kk % NBUF
            hg[b].wait()
            us = ubuf.at[pl.ds(kk * CH, CH)]
            vs = vbuf.at[pl.ds(kk * CH, CH)]
            h1 = pltpu.async_copy(rbuf[b], acc.at[us], ssem[b], add=True)
            h2 = pltpu.async_copy(rbuf[b], acc.at[vs], ssem[b], add=True)
            hs[b] = (h1, h2)
            if kk + NBUF < NFULL:
                h1.wait()
                h2.wait()
                hg[b] = pltpu.async_copy(
                    rows_h.at[pl.ds(base + (kk + NBUF) * CH, CH)],
                    rbuf[b], gsem[b])
        ut = ubuf.at[pl.ds(NFULL * CH, TAIL)]
        vt = vbuf.at[pl.ds(NFULL * CH, TAIL)]
        pltpu.sync_copy(rows_h.at[pl.ds(base + NFULL * CH, TAIL)], rows8)
        pltpu.sync_copy(rows8, acc.at[ut], add=True)
        pltpu.sync_copy(rows8, acc.at[vt], add=True)
        for b in range(NBUF):
            hs[b][0].wait()
            hs[b][1].wait()

        plsc.subcore_barrier()

        @pl.when(cid == 0)
        def _():
            _tile_rows_copy(acc, out0, sid)

        @pl.when(cid == 1)
        def _():
            _tile_rows_copy(acc, out1, sid)

    return k(rows_hbm, u_hbm, v_hbm, zeros_hbm)


def _sc_neighbor_sum(emb_hbm, u_hbm, v_hbm, zeros_hbm):
    @functools.partial(
        pl.kernel,
        out_type=[jax.ShapeDtypeStruct((N, D), _f32),
                  jax.ShapeDtypeStruct((N, D), _f32)],
        mesh=_MESH,
        scratch_types=[
            pltpu.VMEM((EPT,), jnp.int32),
            pltpu.VMEM((EPT,), jnp.int32),
            pltpu.VMEM((CHN, D), _f32),
            pltpu.VMEM((CHN, D), _f32),
            pltpu.VMEM((CHN, D), _f32),
            pltpu.VMEM((CHN, D), _f32),
            pltpu.VMEM((TAIL, D), _f32),
            pltpu.VMEM((TAIL, D), _f32),
            pltpu.VMEM_SHARED((N, D), _f32),
            pltpu.SemaphoreType.DMA,
            pltpu.SemaphoreType.DMA,
            pltpu.SemaphoreType.DMA,
            pltpu.SemaphoreType.DMA,
        ],
    )
    def k(emb_h, u_h, v_h, z_h, out0, out1, ubuf, vbuf, ru0, rv0, ru1, rv1,
          rut, rvt, acc, g0, g1, s0, s1):
        cid = lax.axis_index("c")
        sid = lax.axis_index("s")
        base = cid * EPC + sid * EPT
        rubuf = [ru0, ru1]
        rvbuf = [rv0, rv1]
        gsem = [g0, g1]
        ssem = [s0, s1]

        pltpu.sync_copy(u_h.at[pl.ds(base, EPT)], ubuf)
        pltpu.sync_copy(v_h.at[pl.ds(base, EPT)], vbuf)

        def start_gather(kk, b):
            us = ubuf.at[pl.ds(kk * CHN, CHN)]
            vs = vbuf.at[pl.ds(kk * CHN, CHN)]
            return (pltpu.async_copy(emb_h.at[us], rubuf[b], gsem[b]),
                    pltpu.async_copy(emb_h.at[vs], rvbuf[b], gsem[b]))

        hg = [start_gather(b, b) for b in range(NBUF)]
        _tile_rows_copy(z_h, acc, sid)
        plsc.subcore_barrier()

        hs = [None] * NBUF
        for kk in range(NFULLN):
            b = kk % NBUF
            hg[b][0].wait()
            hg[b][1].wait()
            us = ubuf.at[pl.ds(kk * CHN, CHN)]
            vs = vbuf.at[pl.ds(kk * CHN, CHN)]
            h1 = pltpu.async_copy(rvbuf[b], acc.at[us], ssem[b], add=True)
            h2 = pltpu.async_copy(rubuf[b], acc.at[vs], ssem[b], add=True)
            hs[b] = (h1, h2)
            if kk + NBUF < NFULLN:
                h1.wait()
                h2.wait()
                hg[b] = start_gather(kk + NBUF, b)
        ut = ubuf.at[pl.ds(NFULLN * CHN, TAIL)]
        vt = vbuf.at[pl.ds(NFULLN * CHN, TAIL)]
        pltpu.sync_copy(emb_h.at[ut], rut)
        pltpu.sync_copy(emb_h.at[vt], rvt)
        pltpu.sync_copy(rvt, acc.at[ut], add=True)
        pltpu.sync_copy(rut, acc.at[vt], add=True)
        for b in range(NBUF):
            hs[b][0].wait()
            hs[b][1].wait()

        plsc.subcore_barrier()

        @pl.when(cid == 0)
        def _():
            _tile_rows_copy(acc, out0, sid)

        @pl.when(cid == 1)
        def _():
            _tile_rows_copy(acc, out1, sid)

    return k(emb_hbm, u_hbm, v_hbm, zeros_hbm)


_EPAD = EPT + 16 - EPT % 16
_QG = _EPAD // 16


def _sc_edge_q(a_hbm, b_hbm, c_hbm, u_hbm, v_hbm):
    @functools.partial(
        pl.kernel,
        out_type=jax.ShapeDtypeStruct((E,), _f32),
        mesh=_MESH,
        scratch_types=[
            pltpu.VMEM((16,), _f32),
            pltpu.VMEM((EPT,), jnp.int32),
            pltpu.VMEM((EPT,), jnp.int32),
            pltpu.VMEM((_EPAD,), _f32),
            pltpu.VMEM((_EPAD,), _f32),
            pltpu.VMEM((_EPAD,), _f32),
            pltpu.SemaphoreType.DMA,
        ],
    )
    def k(a_h, b_h, c_h, u_h, v_h, out, cbuf, ubuf, vbuf, av, bv, qbuf, sem):
        cid = lax.axis_index("c")
        sid = lax.axis_index("s")
        tid = cid * NS + sid
        base = tid * EPT
        pltpu.sync_copy(c_h, cbuf)
        pltpu.sync_copy(u_h.at[pl.ds(base, EPT)], ubuf)
        pltpu.sync_copy(v_h.at[pl.ds(base, EPT)], vbuf)
        cp_a = pltpu.async_copy(a_h.at[ubuf], av.at[pl.ds(0, EPT)], sem)
        cp_b = pltpu.async_copy(b_h.at[vbuf], bv.at[pl.ds(0, EPT)], sem)
        cp_a.wait()
        cp_b.wait()
        cv = cbuf[...]

        def body(i, _):
            qbuf[pl.ds(i * 16, 16)] = (
                av[pl.ds(i * 16, 16)] + bv[pl.ds(i * 16, 16)] + cv)
            return 0

        lax.fori_loop(0, _QG, body, 0)
        pltpu.sync_copy(qbuf.at[pl.ds(0, EPT)], out.at[pl.ds(base, EPT)])

    return k(a_hbm, b_hbm, c_hbm, u_hbm, v_hbm)


def kernel(state, edge_features, edges_ij, W1, W2, W3, W4, W5, W6, W7, Wnoop):
    u = edges_ij[:, 0]
    v = edges_ij[:, 1]
    zeros_nd = jnp.zeros((N, D), _f32)

    ne = _mm_rows(state[0], W1, act=False, block_rows=2000)
    x4 = _mm_rows(edge_features[0], W4, act=True, block_rows=4000)

    m0, m1 = _sc_scatter_rows(x4, u, v, zeros_nd)
    efe = _mm_sum2(m0, m1, W3, block_rows=2000)

    emb = ne
    for _ in range(3):
        n0, n1 = _sc_neighbor_sum(emb, u, v, zeros_nd)
        emb = _layer_update(n0, n1, ne, efe, W2, block_rows=2000)

    a, b, g = _post(emb, W7, W5, block_rows=2000)
    c_full, noop = _finalize(g, W6, W5, Wnoop)
    c16 = c_full[0, :16]

    eq = _sc_edge_q(a[:, 0], b[:, 0], c16, u, v)
    return jnp.concatenate([eq[None, :], noop], axis=1)


# speedup vs baseline: 7.5474x; 1.0072x over previous
import functools

import jax
import jax.numpy as jnp
from jax import lax
from jax.experimental import pallas as pl
from jax.experimental.pallas import tpu as pltpu
from jax.experimental.pallas import tpu_sc as plsc

N = 10000
E = 160000
D = 128

NC = 2
NS = 16
EPC = E // NC
EPT = EPC // NS
CH = 40
NFULL = EPT // CH
NBUF = 3
RPT = 624
RREM = N - NS * RPT

_f32 = jnp.float32


def _lrelu(x):
    return jnp.where(x >= 0, x, 0.01 * x)


def _mm_rows(x, w, act, block_rows):
    R, K = x.shape
    C = w.shape[1]

    def body(x_ref, w_ref, o_ref):
        y = jnp.dot(x_ref[...], w_ref[...], preferred_element_type=_f32)
        o_ref[...] = _lrelu(y) if act else y

    return pl.pallas_call(
        body,
        grid=(R // block_rows,),
        in_specs=[
            pl.BlockSpec((block_rows, K), lambda i: (i, 0)),
            pl.BlockSpec((K, C), lambda i: (0, 0)),
        ],
        out_specs=pl.BlockSpec((block_rows, C), lambda i: (i, 0)),
        out_shape=jax.ShapeDtypeStruct((R, C), _f32),
    )(x, w)


def _mm_sum2(m0, m1, w, block_rows):
    R, K = m0.shape
    C = w.shape[1]

    def body(a_ref, b_ref, w_ref, o_ref):
        s = a_ref[...] + b_ref[...]
        o_ref[...] = jnp.dot(s, w_ref[...], preferred_element_type=_f32)

    spec = pl.BlockSpec((block_rows, K), lambda i: (i, 0))
    return pl.pallas_call(
        body,
        grid=(R // block_rows,),
        in_specs=[spec, spec, pl.BlockSpec((K, C), lambda i: (0, 0))],
        out_specs=pl.BlockSpec((block_rows, C), lambda i: (i, 0)),
        out_shape=jax.ShapeDtypeStruct((R, C), _f32),
    )(m0, m1, w)


def _layer_update(n0, n1, ne, efe, w2, block_rows):
    R, K = ne.shape

    def body(n0_ref, n1_ref, ne_ref, efe_ref, w_ref, o_ref):
        s = n0_ref[...] + n1_ref[...]
        y = ne_ref[...] + efe_ref[...] + jnp.dot(
            s, w_ref[...], preferred_element_type=_f32)
        o_ref[...] = _lrelu(y)

    spec = pl.BlockSpec((block_rows, K), lambda i: (i, 0))
    return pl.pallas_call(
        body,
        grid=(R // block_rows,),
        in_specs=[spec, spec, spec, spec,
                  pl.BlockSpec((K, K), lambda i: (0, 0))],
        out_specs=pl.BlockSpec((block_rows, K), lambda i: (i, 0)),
        out_shape=jax.ShapeDtypeStruct((R, K), _f32),
    )(n0, n1, ne, efe, w2)


def _post(emb, w7, w5, block_rows):
    R = emb.shape[0]

    def body(emb_ref, w7_ref, w5_ref, a_ref, b_ref, g_ref):
        i = pl.program_id(0)
        npj = jnp.dot(emb_ref[...], w7_ref[...], preferred_element_type=_f32)
        lr = _lrelu(npj)
        a_ref[...] = jnp.dot(lr, w5_ref[D:2 * D, :],
                             preferred_element_type=_f32)
        b_ref[...] = jnp.dot(lr, w5_ref[2 * D:3 * D, :],
                             preferred_element_type=_f32)

        @pl.when(i == 0)
        def _():
            g_ref[...] = jnp.zeros_like(g_ref)

        g_ref[...] += jnp.sum(emb_ref[...], axis=0, keepdims=True)

    return pl.pallas_call(
        body,
        grid=(R // block_rows,),
        in_specs=[
            pl.BlockSpec((block_rows, D), lambda i: (i, 0)),
            pl.BlockSpec((D, D), lambda i: (0, 0)),
            pl.BlockSpec((3 * D, 1), lambda i: (0, 0)),
        ],
        out_specs=[
            pl.BlockSpec((block_rows, 1), lambda i: (i, 0)),
            pl.BlockSpec((block_rows, 1), lambda i: (i, 0)),
            pl.BlockSpec((1, D), lambda i: (0, 0)),
        ],
        out_shape=[
            jax.ShapeDtypeStruct((R, 1), _f32),
            jax.ShapeDtypeStruct((R, 1), _f32),
            jax.ShapeDtypeStruct((1, D), _f32),
        ],
    )(emb, w7, w5)


def _finalize(g, w6, w5, wnoop):
    def body(g_ref, w6_ref, w5_ref, wn_ref, c_ref, noop_ref):
        gv = g_ref[...]
        lr = _lrelu(jnp.dot(gv, w6_ref[...], preferred_element_type=_f32))
        c = jnp.dot(lr, w5_ref[0:D, :], preferred_element_type=_f32)
        c_ref[...] = jnp.broadcast_to(c, c_ref.shape)
        noop_ref[...] = jnp.dot(gv, wn_ref[...], preferred_element_type=_f32)

    return pl.pallas_call(
        body,
        out_shape=[
            jax.ShapeDtypeStruct((1, D), _f32),
            jax.ShapeDtypeStruct((1, 1), _f32),
        ],
    )(g, w6, w5, wnoop)


_MESH = plsc.VectorSubcoreMesh(core_axis_name="c", subcore_axis_name="s")


def _tile_rows_copy(src, dst, sid):
    r0 = sid * RPT
    pltpu.sync_copy(src.at[pl.ds(r0, RPT)], dst.at[pl.ds(r0, RPT)])

    @pl.when(sid == 0)
    def _():
        pltpu.sync_copy(src.at[pl.ds(NS * RPT, RREM)],
                        dst.at[pl.ds(NS * RPT, RREM)])


def _sc_scatter_rows(rows_hbm, u_hbm, v_hbm, zeros_hbm):
    @functools.partial(
        pl.kernel,
        out_type=[jax.ShapeDtypeStruct((N, D), _f32),
                  jax.ShapeDtypeStruct((N, D), _f32)],
        mesh=_MESH,
        scratch_types=[
            pltpu.VMEM((EPT,), jnp.int32),
            pltpu.VMEM((EPT,), jnp.int32),
            pltpu.VMEM((CH, D), _f32),
            pltpu.VMEM((CH, D), _f32),
            pltpu.VMEM((CH, D), _f32),
            pltpu.VMEM_SHARED((N, D), _f32),
            pltpu.SemaphoreType.DMA,
            pltpu.SemaphoreType.DMA,
            pltpu.SemaphoreType.DMA,
            pltpu.SemaphoreType.DMA,
            pltpu.SemaphoreType.DMA,
            pltpu.SemaphoreType.DMA,
        ],
    )
    def k(rows_h, u_h, v_h, z_h, out0, out1, ubuf, vbuf, r0, r1, r2,
          acc, g0, g1, g2, s0, s1, s2):
        cid = lax.axis_index("c")
        sid = lax.axis_index("s")
        base = cid * EPC + sid * EPT
        rbuf = [r0, r1, r2]
        gsem = [g0, g1, g2]
        ssem = [s0, s1, s2]

        pltpu.sync_copy(u_h.at[pl.ds(base, EPT)], ubuf)
        pltpu.sync_copy(v_h.at[pl.ds(base, EPT)], vbuf)
        hg = [pltpu.async_copy(rows_h.at[pl.ds(base + b * CH, CH)],
                               rbuf[b], gsem[b]) for b in range(NBUF)]
        _tile_rows_copy(z_h, acc, sid)
        plsc.subcore_barrier()

        hs = [None] * NBUF
        for kk in range(NFULL):
            b = kk % NBUF
            hg[b].wait()
            us = ubuf.at[pl.ds(kk * CH, CH)]
            vs = vbuf.at[pl.ds(kk * CH, CH)]
            h1 = pltpu.async_copy(rbuf[b], acc.at[us], ssem[b], add=True)
            h2 = pltpu.async_copy(rbuf[b], acc.at[vs], ssem[b], add=True)
            hs[b] = (h1, h2)
            if kk >= 1 and kk + 2 < NFULL:
                bp = (kk + 2) % NBUF
                hs[bp][0].wait()
                hs[bp][1].wait()
                hg[bp] = pltpu.async_copy(
                    rows_h.at[pl.ds(base + (kk + 2) * CH, CH)],
                    rbuf[bp], gsem[bp])
        for kk in range(NFULL - NBUF, NFULL):
            b = kk % NBUF
            hs[b][0].wait()
            hs[b][1].wait()

        plsc.subcore_barrier()

        @pl.when(cid == 0)
        def _():
            _tile_rows_copy(acc, out0, sid)

        @pl.when(cid == 1)
        def _():
            _tile_rows_copy(acc, out1, sid)

    return k(rows_hbm, u_hbm, v_hbm, zeros_hbm)


def _sc_neighbor_sum(emb_hbm, u_hbm, v_hbm, zeros_hbm):
    @functools.partial(
        pl.kernel,
        out_type=[jax.ShapeDtypeStruct((N, D), _f32),
                  jax.ShapeDtypeStruct((N, D), _f32)],
        mesh=_MESH,
        scratch_types=[
            pltpu.VMEM((EPT,), jnp.int32),
            pltpu.VMEM((EPT,), jnp.int32),
            pltpu.VMEM((CH, D), _f32),
            pltpu.VMEM((CH, D), _f32),
            pltpu.VMEM((CH, D), _f32),
            pltpu.VMEM((CH, D), _f32),
            pltpu.VMEM((CH, D), _f32),
            pltpu.VMEM((CH, D), _f32),
            pltpu.VMEM_SHARED((N, D), _f32),
            pltpu.SemaphoreType.DMA,
            pltpu.SemaphoreType.DMA,
            pltpu.SemaphoreType.DMA,
            pltpu.SemaphoreType.DMA,
            pltpu.SemaphoreType.DMA,
            pltpu.SemaphoreType.DMA,
        ],
    )
    def k(emb_h, u_h, v_h, z_h, out0, out1, ubuf, vbuf, ru0, rv0, ru1, rv1,
          ru2, rv2, acc, g0, g1, g2, s0, s1, s2):
        cid = lax.axis_index("c")
        sid = lax.axis_index("s")
        base = cid * EPC + sid * EPT
        rubuf = [ru0, ru1, ru2]
        rvbuf = [rv0, rv1, rv2]
        gsem = [g0, g1, g2]
        ssem = [s0, s1, s2]

        pltpu.sync_copy(u_h.at[pl.ds(base, EPT)], ubuf)
        pltpu.sync_copy(v_h.at[pl.ds(base, EPT)], vbuf)

        def start_gather(kk, b):
            us = ubuf.at[pl.ds(kk * CH, CH)]
            vs = vbuf.at[pl.ds(kk * CH, CH)]
            return (pltpu.async_copy(emb_h.at[us], rubuf[b], gsem[b]),
                    pltpu.async_copy(emb_h.at[vs], rvbuf[b], gsem[b]))

        hg = [start_gather(b, b) for b in range(NBUF)]
        _tile_rows_copy(z_h, acc, sid)
        plsc.subcore_barrier()

        hs = [None] * NBUF
        for kk in range(NFULL):
            b = kk % NBUF
            hg[b][0].wait()
            hg[b][1].wait()
            us = ubuf.at[pl.ds(kk * CH, CH)]
            vs = vbuf.at[pl.ds(kk * CH, CH)]
            h1 = pltpu.async_copy(rvbuf[b], acc.at[us], ssem[b], add=True)
            h2 = pltpu.async_copy(rubuf[b], acc.at[vs], ssem[b], add=True)
            hs[b] = (h1, h2)
            if kk >= 1 and kk + 2 < NFULL:
                bp = (kk + 2) % NBUF
                hs[bp][0].wait()
                hs[bp][1].wait()
                hg[bp] = start_gather(kk + 2, bp)
        for kk in range(NFULL - NBUF, NFULL):
            b = kk % NBUF
            hs[b][0].wait()
            hs[b][1].wait()

        plsc.subcore_barrier()

        @pl.when(cid == 0)
        def _():
            _tile_rows_copy(acc, out0, sid)

        @pl.when(cid == 1)
        def _():
            _tile_rows_copy(acc, out1, sid)

    return k(emb_hbm, u_hbm, v_hbm, zeros_hbm)


_EPAD = EPT + 16 - EPT % 16
_QG = _EPAD // 16


def _sc_edge_q(a_hbm, b_hbm, c_hbm, u_hbm, v_hbm):
    @functools.partial(
        pl.kernel,
        out_type=jax.ShapeDtypeStruct((E,), _f32),
        mesh=_MESH,
        scratch_types=[
            pltpu.VMEM((16,), _f32),
            pltpu.VMEM((EPT,), jnp.int32),
            pltpu.VMEM((EPT,), jnp.int32),
            pltpu.VMEM((_EPAD,), _f32),
            pltpu.VMEM((_EPAD,), _f32),
            pltpu.VMEM((_EPAD,), _f32),
            pltpu.SemaphoreType.DMA,
        ],
    )
    def k(a_h, b_h, c_h, u_h, v_h, out, cbuf, ubuf, vbuf, av, bv, qbuf, sem):
        cid = lax.axis_index("c")
        sid = lax.axis_index("s")
        tid = cid * NS + sid
        base = tid * EPT
        pltpu.sync_copy(c_h, cbuf)
        pltpu.sync_copy(u_h.at[pl.ds(base, EPT)], ubuf)
        pltpu.sync_copy(v_h.at[pl.ds(base, EPT)], vbuf)
        cp_a = pltpu.async_copy(a_h.at[ubuf], av.at[pl.ds(0, EPT)], sem)
        cp_b = pltpu.async_copy(b_h.at[vbuf], bv.at[pl.ds(0, EPT)], sem)
        cp_a.wait()
        cp_b.wait()
        cv = cbuf[...]

        def body(i, _):
            qbuf[pl.ds(i * 16, 16)] = (
                av[pl.ds(i * 16, 16)] + bv[pl.ds(i * 16, 16)] + cv)
            return 0

        lax.fori_loop(0, _QG, body, 0)
        pltpu.sync_copy(qbuf.at[pl.ds(0, EPT)], out.at[pl.ds(base, EPT)])

    return k(a_hbm, b_hbm, c_hbm, u_hbm, v_hbm)


def kernel(state, edge_features, edges_ij, W1, W2, W3, W4, W5, W6, W7, Wnoop):
    u = edges_ij[:, 0]
    v = edges_ij[:, 1]
    zeros_nd = jnp.zeros((N, D), _f32)

    ne = _mm_rows(state[0], W1, act=False, block_rows=2000)
    x4 = _mm_rows(edge_features[0], W4, act=True, block_rows=4000)

    m0, m1 = _sc_scatter_rows(x4, u, v, zeros_nd)
    efe = _mm_sum2(m0, m1, W3, block_rows=2000)

    emb = ne
    for _ in range(3):
        n0, n1 = _sc_neighbor_sum(emb, u, v, zeros_nd)
        emb = _layer_update(n0, n1, ne, efe, W2, block_rows=2000)

    a, b, g = _post(emb, W7, W5, block_rows=2000)
    c_full, noop = _finalize(g, W6, W5, Wnoop)
    c16 = c_full[0, :16]

    eq = _sc_edge_q(a[:, 0], b[:, 0], c16, u, v)
    return jnp.concatenate([eq[None, :], noop], axis=1)


# speedup vs baseline: 7.6204x; 1.0097x over previous
import functools

import jax
import jax.numpy as jnp
from jax import lax
from jax.experimental import pallas as pl
from jax.experimental.pallas import tpu as pltpu
from jax.experimental.pallas import tpu_sc as plsc

N = 10000
E = 160000
D = 128

NC = 2
NS = 16
EPC = E // NC
EPT = EPC // NS
CH = 40
NFULL = EPT // CH
NBUF = 3
CHS = 96
NFULLS = EPT // CHS
TAILS = EPT - NFULLS * CHS
RPT = 624
RREM = N - NS * RPT

_f32 = jnp.float32


def _lrelu(x):
    return jnp.where(x >= 0, x, 0.01 * x)


def _mm_rows(x, w, act, block_rows):
    R, K = x.shape
    C = w.shape[1]

    def body(x_ref, w_ref, o_ref):
        y = jnp.dot(x_ref[...], w_ref[...], preferred_element_type=_f32)
        o_ref[...] = _lrelu(y) if act else y

    return pl.pallas_call(
        body,
        grid=(R // block_rows,),
        in_specs=[
            pl.BlockSpec((block_rows, K), lambda i: (i, 0)),
            pl.BlockSpec((K, C), lambda i: (0, 0)),
        ],
        out_specs=pl.BlockSpec((block_rows, C), lambda i: (i, 0)),
        out_shape=jax.ShapeDtypeStruct((R, C), _f32),
    )(x, w)


def _mm_sum3(ne, m0, m1, w, block_rows):
    R, K = m0.shape
    C = w.shape[1]

    def body(ne_ref, a_ref, b_ref, w_ref, o_ref):
        s = a_ref[...] + b_ref[...]
        o_ref[...] = ne_ref[...] + jnp.dot(
            s, w_ref[...], preferred_element_type=_f32)

    spec = pl.BlockSpec((block_rows, K), lambda i: (i, 0))
    return pl.pallas_call(
        body,
        grid=(R // block_rows,),
        in_specs=[spec, spec, spec, pl.BlockSpec((K, C), lambda i: (0, 0))],
        out_specs=pl.BlockSpec((block_rows, C), lambda i: (i, 0)),
        out_shape=jax.ShapeDtypeStruct((R, C), _f32),
    )(ne, m0, m1, w)


def _layer_update(n0, n1, base, w2, block_rows):
    R, K = base.shape

    def body(n0_ref, n1_ref, base_ref, w_ref, o_ref):
        s = n0_ref[...] + n1_ref[...]
        y = base_ref[...] + jnp.dot(s, w_ref[...],
                                    preferred_element_type=_f32)
        o_ref[...] = _lrelu(y)

    spec = pl.BlockSpec((block_rows, K), lambda i: (i, 0))
    return pl.pallas_call(
        body,
        grid=(R // block_rows,),
        in_specs=[spec, spec, spec,
                  pl.BlockSpec((K, K), lambda i: (0, 0))],
        out_specs=pl.BlockSpec((block_rows, K), lambda i: (i, 0)),
        out_shape=jax.ShapeDtypeStruct((R, K), _f32),
    )(n0, n1, base, w2)


def _post(emb, w7, w5, w6, wnoop, block_rows):
    R = emb.shape[0]
    nblk = R // block_rows

    def body(emb_ref, w7_ref, w5_ref, w6_ref, wn_ref,
             a_ref, b_ref, c_ref, noop_ref, g_ref):
        i = pl.program_id(0)
        npj = jnp.dot(emb_ref[...], w7_ref[...], preferred_element_type=_f32)
        lr = _lrelu(npj)
        a_ref[...] = jnp.dot(lr, w5_ref[D:2 * D, :],
                             preferred_element_type=_f32)
        b_ref[...] = jnp.dot(lr, w5_ref[2 * D:3 * D, :],
                             preferred_element_type=_f32)

        @pl.when(i == 0)
        def _():
            g_ref[...] = jnp.zeros_like(g_ref)

        g_ref[...] += jnp.sum(emb_ref[...], axis=0, keepdims=True)

        @pl.when(i == nblk - 1)
        def _():
            gv = g_ref[...]
            glr = _lrelu(jnp.dot(gv, w6_ref[...], preferred_element_type=_f32))
            c = jnp.dot(glr, w5_ref[0:D, :], preferred_element_type=_f32)
            c_ref[...] = jnp.broadcast_to(c, c_ref.shape)
            noop_ref[...] = jnp.dot(gv, wn_ref[...],
                                    preferred_element_type=_f32)

    return pl.pallas_call(
        body,
        grid=(nblk,),
        in_specs=[
            pl.BlockSpec((block_rows, D), lambda i: (i, 0)),
            pl.BlockSpec((D, D), lambda i: (0, 0)),
            pl.BlockSpec((3 * D, 1), lambda i: (0, 0)),
            pl.BlockSpec((D, D), lambda i: (0, 0)),
            pl.BlockSpec((D, 1), lambda i: (0, 0)),
        ],
        out_specs=[
            pl.BlockSpec((block_rows, 1), lambda i: (i, 0)),
            pl.BlockSpec((block_rows, 1), lambda i: (i, 0)),
            pl.BlockSpec((1, 16), lambda i: (0, 0)),
            pl.BlockSpec((1, 1), lambda i: (0, 0)),
        ],
        out_shape=[
            jax.ShapeDtypeStruct((R, 1), _f32),
            jax.ShapeDtypeStruct((R, 1), _f32),
            jax.ShapeDtypeStruct((1, 16), _f32),
            jax.ShapeDtypeStruct((1, 1), _f32),
        ],
        scratch_shapes=[pltpu.VMEM((1, D), _f32)],
    )(emb, w7, w5, w6, wnoop)


_MESH = plsc.VectorSubcoreMesh(core_axis_name="c", subcore_axis_name="s")


def _tile_rows_copy(src, dst, sid):
    r0 = sid * RPT
    pltpu.sync_copy(src.at[pl.ds(r0, RPT)], dst.at[pl.ds(r0, RPT)])

    @pl.when(sid == 0)
    def _():
        pltpu.sync_copy(src.at[pl.ds(NS * RPT, RREM)],
                        dst.at[pl.ds(NS * RPT, RREM)])


def _sc_scatter_rows(rows_hbm, u_hbm, v_hbm, zeros_hbm):
    @functools.partial(
        pl.kernel,
        out_type=[jax.ShapeDtypeStruct((N, D), _f32),
                  jax.ShapeDtypeStruct((N, D), _f32)],
        mesh=_MESH,
        scratch_types=[
            pltpu.VMEM((EPT,), jnp.int32),
            pltpu.VMEM((EPT,), jnp.int32),
            pltpu.VMEM((CHS, D), _f32),
            pltpu.VMEM((CHS, D), _f32),
            pltpu.VMEM((CHS, D), _f32),
            pltpu.VMEM((TAILS, D), _f32),
            pltpu.VMEM_SHARED((N, D), _f32),
            pltpu.SemaphoreType.DMA,
            pltpu.SemaphoreType.DMA,
            pltpu.SemaphoreType.DMA,
            pltpu.SemaphoreType.DMA,
            pltpu.SemaphoreType.DMA,
            pltpu.SemaphoreType.DMA,
        ],
    )
    def k(rows_h, u_h, v_h, z_h, out0, out1, ubuf, vbuf, r0, r1, r2,
          rowst, acc, g0, g1, g2, s0, s1, s2):
        cid = lax.axis_index("c")
        sid = lax.axis_index("s")
        base = cid * EPC + sid * EPT
        rbuf = [r0, r1, r2]
        gsem = [g0, g1, g2]
        ssem = [s0, s1, s2]

        pltpu.sync_copy(u_h.at[pl.ds(base, EPT)], ubuf)
        pltpu.sync_copy(v_h.at[pl.ds(base, EPT)], vbuf)
        hg = [pltpu.async_copy(rows_h.at[pl.ds(base + b * CHS, CHS)],
                               rbuf[b], gsem[b]) for b in range(NBUF)]
        _tile_rows_copy(z_h, acc, sid)
        plsc.subcore_barrier()

        hs = [None] * NBUF
        for kk in range(NFULLS):
            b = kk % NBUF
            hg[b].wait()
            us = ubuf.at[pl.ds(kk * CHS, CHS)]
            vs = vbuf.at[pl.ds(kk * CHS, CHS)]
            h1 = pltpu.async_copy(rbuf[b], acc.at[us], ssem[b], add=True)
            h2 = pltpu.async_copy(rbuf[b], acc.at[vs], ssem[b], add=True)
            hs[b] = (h1, h2)
            if kk >= 1 and kk + 2 < NFULLS:
                bp = (kk + 2) % NBUF
                hs[bp][0].wait()
                hs[bp][1].wait()
                hg[bp] = pltpu.async_copy(
                    rows_h.at[pl.ds(base + (kk + 2) * CHS, CHS)],
                    rbuf[bp], gsem[bp])
        ut = ubuf.at[pl.ds(NFULLS * CHS, TAILS)]
        vt = vbuf.at[pl.ds(NFULLS * CHS, TAILS)]
        pltpu.sync_copy(rows_h.at[pl.ds(base + NFULLS * CHS, TAILS)], rowst)
        pltpu.sync_copy(rowst, acc.at[ut], add=True)
        pltpu.sync_copy(rowst, acc.at[vt], add=True)
        for kk in range(NFULLS - NBUF, NFULLS):
            b = kk % NBUF
            hs[b][0].wait()
            hs[b][1].wait()

        plsc.subcore_barrier()

        @pl.when(cid == 0)
        def _():
            _tile_rows_copy(acc, out0, sid)

        @pl.when(cid == 1)
        def _():
            _tile_rows_copy(acc, out1, sid)

    return k(rows_hbm, u_hbm, v_hbm, zeros_hbm)


def _sc_neighbor_sum(emb_hbm, u_hbm, v_hbm, zeros_hbm):
    @functools.partial(
        pl.kernel,
        out_type=[jax.ShapeDtypeStruct((N, D), _f32),
                  jax.ShapeDtypeStruct((N, D), _f32)],
        mesh=_MESH,
        scratch_types=[
            pltpu.VMEM((EPT,), jnp.int32),
            pltpu.VMEM((EPT,), jnp.int32),
            pltpu.VMEM((CH, D), _f32),
            pltpu.VMEM((CH, D), _f32),
            pltpu.VMEM((CH, D), _f32),
            pltpu.VMEM((CH, D), _f32),
            pltpu.VMEM((CH, D), _f32),
            pltpu.VMEM((CH, D), _f32),
            pltpu.VMEM_SHARED((N, D), _f32),
            pltpu.SemaphoreType.DMA,
            pltpu.SemaphoreType.DMA,
            pltpu.SemaphoreType.DMA,
            pltpu.SemaphoreType.DMA,
            pltpu.SemaphoreType.DMA,
            pltpu.SemaphoreType.DMA,
        ],
    )
    def k(emb_h, u_h, v_h, z_h, out0, out1, ubuf, vbuf, ru0, rv0, ru1, rv1,
          ru2, rv2, acc, g0, g1, g2, s0, s1, s2):
        cid = lax.axis_index("c")
        sid = lax.axis_index("s")
        base = cid * EPC + sid * EPT
        rubuf = [ru0, ru1, ru2]
        rvbuf = [rv0, rv1, rv2]
        gsem = [g0, g1, g2]
        ssem = [s0, s1, s2]

        pltpu.sync_copy(u_h.at[pl.ds(base, EPT)], ubuf)
        pltpu.sync_copy(v_h.at[pl.ds(base, EPT)], vbuf)

        def start_gather(kk, b):
            us = ubuf.at[pl.ds(kk * CH, CH)]
            vs = vbuf.at[pl.ds(kk * CH, CH)]
            return (pltpu.async_copy(emb_h.at[us], rubuf[b], gsem[b]),
                    pltpu.async_copy(emb_h.at[vs], rvbuf[b], gsem[b]))

        hg = [start_gather(b, b) for b in range(NBUF)]
        _tile_rows_copy(z_h, acc, sid)
        plsc.subcore_barrier()

        hs = [None] * NBUF
        for kk in range(NFULL):
            b = kk % NBUF
            hg[b][0].wait()
            hg[b][1].wait()
            us = ubuf.at[pl.ds(kk * CH, CH)]
            vs = vbuf.at[pl.ds(kk * CH, CH)]
            h1 = pltpu.async_copy(rvbuf[b], acc.at[us], ssem[b], add=True)
            h2 = pltpu.async_copy(rubuf[b], acc.at[vs], ssem[b], add=True)
            hs[b] = (h1, h2)
            if kk >= 1 and kk + 2 < NFULL:
                bp = (kk + 2) % NBUF
                hs[bp][0].wait()
                hs[bp][1].wait()
                hg[bp] = start_gather(kk + 2, bp)
        for kk in range(NFULL - NBUF, NFULL):
            b = kk % NBUF
            hs[b][0].wait()
            hs[b][1].wait()

        plsc.subcore_barrier()

        @pl.when(cid == 0)
        def _():
            _tile_rows_copy(acc, out0, sid)

        @pl.when(cid == 1)
        def _():
            _tile_rows_copy(acc, out1, sid)

    return k(emb_hbm, u_hbm, v_hbm, zeros_hbm)


_EPAD = EPT + 16 - EPT % 16
_QG = _EPAD // 16


def _sc_edge_q(a_hbm, b_hbm, c_hbm, u_hbm, v_hbm):
    @functools.partial(
        pl.kernel,
        out_type=jax.ShapeDtypeStruct((E,), _f32),
        mesh=_MESH,
        scratch_types=[
            pltpu.VMEM((16,), _f32),
            pltpu.VMEM((EPT,), jnp.int32),
            pltpu.VMEM((EPT,), jnp.int32),
            pltpu.VMEM((_EPAD,), _f32),
            pltpu.VMEM((_EPAD,), _f32),
            pltpu.VMEM((_EPAD,), _f32),
            pltpu.SemaphoreType.DMA,
        ],
    )
    def k(a_h, b_h, c_h, u_h, v_h, out, cbuf, ubuf, vbuf, av, bv, qbuf, sem):
        cid = lax.axis_index("c")
        sid = lax.axis_index("s")
        tid = cid * NS + sid
        base = tid * EPT
        pltpu.sync_copy(c_h, cbuf)
        pltpu.sync_copy(u_h.at[pl.ds(base, EPT)], ubuf)
        pltpu.sync_copy(v_h.at[pl.ds(base, EPT)], vbuf)
        cp_a = pltpu.async_copy(a_h.at[ubuf], av.at[pl.ds(0, EPT)], sem)
        cp_b = pltpu.async_copy(b_h.at[vbuf], bv.at[pl.ds(0, EPT)], sem)
        cp_a.wait()
        cp_b.wait()
        cv = cbuf[...]

        def body(i, _):
            qbuf[pl.ds(i * 16, 16)] = (
                av[pl.ds(i * 16, 16)] + bv[pl.ds(i * 16, 16)] + cv)
            return 0

        lax.fori_loop(0, _QG, body, 0)
        pltpu.sync_copy(qbuf.at[pl.ds(0, EPT)], out.at[pl.ds(base, EPT)])

    return k(a_hbm, b_hbm, c_hbm, u_hbm, v_hbm)


def kernel(state, edge_features, edges_ij, W1, W2, W3, W4, W5, W6, W7, Wnoop):
    u = edges_ij[:, 0]
    v = edges_ij[:, 1]
    zeros_nd = jnp.zeros((N, D), _f32)

    ne = _mm_rows(state[0], W1, act=False, block_rows=2000)
    x4 = _mm_rows(edge_features[0], W4, act=True, block_rows=4000)

    m0, m1 = _sc_scatter_rows(x4, u, v, zeros_nd)
    base = _mm_sum3(ne, m0, m1, W3, block_rows=2000)

    emb = ne
    for _ in range(3):
        n0, n1 = _sc_neighbor_sum(emb, u, v, zeros_nd)
        emb = _layer_update(n0, n1, base, W2, block_rows=2000)

    a, b, c, noop = _post(emb, W7, W5, W6, Wnoop, block_rows=2000)

    eq = _sc_edge_q(a[:, 0], b[:, 0], c[0], u, v)
    return jnp.concatenate([eq[None, :], noop], axis=1)
